# baseline (speedup 1.0000x reference)
import functools
import math

import numpy as np
import jax
import jax.numpy as jnp
from jax import lax
from jax.experimental import pallas as pl
from jax.experimental.pallas import tpu as pltpu
from jax.experimental.pallas import tpu_sc as plsc

_N = 2048
_H = 8
_N_SPATIAL = 32
_TAB = (2 * 16 + 1) * _N_SPATIAL
_L = 16
_NW = 32
_ROWS_PER_W = _N // _NW
_VECS = _N // _L


def _spatial_d2_thresholds():
    log_c = np.log(np.float32(257.0)).astype(np.float32)
    bins = np.exp(np.linspace(np.float32(0.0), log_c, _N_SPATIAL,
                              dtype=np.float32)).astype(np.float32)

    def mk(b):
        x = np.float32(np.float64(b) ** 2)
        for _ in range(8):
            x = np.nextafter(x, np.float32(-1), dtype=np.float32)
        while not (np.float32(np.sqrt(x)) > b):
            x = np.nextafter(x, np.float32(np.inf), dtype=np.float32)
        return x

    return [float(mk(b)) for b in bins[:_N_SPATIAL - 1]]


_M_THRESH = _spatial_d2_thresholds()
_PAD = 3.0e38
_MT0 = _M_THRESH + [_PAD]
_MT1 = _M_THRESH[1:] + [_PAD, _PAD]
_C = 2.0 * math.log2(257.0) / 31.0
_INV = float(np.float32(1.0 / (_C * 2.0 ** 23)))
_FBIAS = 127 << 23

_mesh = plsc.VectorSubcoreMesh(core_axis_name="c", subcore_axis_name="s")


@functools.partial(
    pl.kernel,
    out_type=jax.ShapeDtypeStruct((_H, _N, _N), jnp.float32),
    mesh=_mesh,
    compiler_params=pltpu.CompilerParams(use_tc_tiling_on_sc=False,
                                          needs_layout_passes=False),
    scratch_types=[
        pltpu.VMEM((_H * _TAB,), jnp.float32),
        pltpu.VMEM((_N + _L,), jnp.float32),
        pltpu.VMEM((_N + _L,), jnp.float32),
        pltpu.VMEM((_N + _L,), jnp.float32),
        pltpu.VMEM((2 * _N_SPATIAL,), jnp.float32),
        pltpu.VMEM((_H, _N), jnp.float32),
        pltpu.VMEM((_H, _N), jnp.float32),
        pltpu.SemaphoreType.DMA,
        pltpu.SemaphoreType.DMA,
    ],
)
def _sc_mask_kernel(coords_t, bias_t, mthr, out, tab, ff, px, py, mt,
                    ob0, ob1, sem0, sem1):
    wid = lax.axis_index("s") * 2 + lax.axis_index("c")
    base = wid * _ROWS_PER_W

    pltpu.sync_copy(bias_t, tab)
    pltpu.sync_copy(coords_t.at[0], ff.at[pl.ds(0, _N)])
    pltpu.sync_copy(coords_t.at[1], px.at[pl.ds(0, _N)])
    pltpu.sync_copy(coords_t.at[2], py.at[pl.ds(0, _N)])
    pltpu.sync_copy(mthr, mt)

    def compute_row(i, ob):
        fi = jnp.full((_L,), ff[pl.ds(i, _L)][0], jnp.float32)
        xi = jnp.full((_L,), px[pl.ds(i, _L)][0], jnp.float32)
        yi = jnp.full((_L,), py[pl.ds(i, _L)][0], jnp.float32)

        def vec_body(v, c):
            o = v * _L
            xj = px[pl.ds(o, _L)]
            yj = py[pl.ds(o, _L)]
            fj = ff[pl.ds(o, _L)]
            dx = xj - xi
            dy = yj - yi
            d2 = dx * dx + dy * dy
            bits = plsc.bitcast(d2, jnp.int32)
            u = (bits - _FBIAS).astype(jnp.float32) * _INV
            kc = jnp.clip(u.astype(jnp.int32), 0, 29)
            m0 = plsc.load_gather(mt, [kc])
            m1 = plsc.load_gather(mt, [kc + _N_SPATIAL])
            s = kc + ((d2 >= m0).astype(jnp.int32)
                      + (d2 >= m1).astype(jnp.int32))
            t_diff = fj - fi
            yhalf = jnp.clip(t_diff * 0.5, -17.0, 17.0)
            tr = yhalf.astype(jnp.int32)
            ceil = tr + jnp.where(tr.astype(jnp.float32) * 2.0 < t_diff, 1, 0)
            tbin = jnp.clip(ceil + 16, 0, 32)
            idx = s + tbin * _N_SPATIAL
            for h in range(_H):
                ob[h, pl.ds(o, _L)] = plsc.load_gather(tab, [idx + h * _TAB])
            return c

        lax.fori_loop(0, _VECS, vec_body, 0)

    def fire(i, ob, sem):
        for h in range(_H):
            pltpu.async_copy(ob.at[h], out.at[h, i], sem)

    def drain(i, ob, sem):
        for h in range(_H):
            pltpu.make_async_copy(ob.at[h], out.at[h, i], sem).wait()

    compute_row(base, ob0)
    fire(base, ob0, sem0)
    compute_row(base + 1, ob1)
    fire(base + 1, ob1, sem1)

    def pair_body(rr, carry):
        i0 = base + 2 * rr
        drain(i0, ob0, sem0)
        compute_row(i0, ob0)
        fire(i0, ob0, sem0)
        i1 = i0 + 1
        drain(i1, ob1, sem1)
        compute_row(i1, ob1)
        fire(i1, ob1, sem1)
        return carry

    lax.fori_loop(1, _ROWS_PER_W // 2, pair_body, 0)

    drain(base + _ROWS_PER_W - 2, ob0, sem0)
    drain(base + _ROWS_PER_W - 1, ob1, sem1)


_MT_ARR = np.array(_MT0 + _MT1, dtype=np.float32)


def kernel(coords, bias):
    coords_t = coords.T
    bias_t = bias.T.reshape(-1)
    return _sc_mask_kernel(coords_t, bias_t, jnp.asarray(_MT_ARR))


# speedup vs baseline: 1.7868x; 1.7868x over previous
import functools
import math

import numpy as np
import jax
import jax.numpy as jnp
from jax import lax
from jax.experimental import pallas as pl
from jax.experimental.pallas import tpu as pltpu
from jax.experimental.pallas import tpu_sc as plsc

_N = 2048
_H = 8
_N_SPATIAL = 32
_TAB = (2 * 16 + 1) * _N_SPATIAL
_L = 16
_NW = 32
_ROWS_PER_W = _N // _NW
_VECS = _N // _L


def _spatial_d2_thresholds():
    log_c = np.log(np.float32(257.0)).astype(np.float32)
    bins = np.exp(np.linspace(np.float32(0.0), log_c, _N_SPATIAL,
                              dtype=np.float32)).astype(np.float32)

    def mk(b):
        x = np.float32(np.float64(b) ** 2)
        for _ in range(8):
            x = np.nextafter(x, np.float32(-1), dtype=np.float32)
        while not (np.float32(np.sqrt(x)) > b):
            x = np.nextafter(x, np.float32(np.inf), dtype=np.float32)
        return x

    return [float(mk(b)) for b in bins[:_N_SPATIAL - 1]]


_M_THRESH = _spatial_d2_thresholds()
_PAD = 3.0e38
_MT0 = _M_THRESH + [_PAD]
_MT1 = _M_THRESH[1:] + [_PAD, _PAD]
_C = 2.0 * math.log2(257.0) / 31.0
_INV = float(np.float32(1.0 / (_C * 2.0 ** 23)))
_FBIAS = 127 << 23

_mesh = plsc.VectorSubcoreMesh(core_axis_name="c", subcore_axis_name="s")


@functools.partial(
    pl.kernel,
    out_type=jax.ShapeDtypeStruct((_H, _N, _N), jnp.float32),
    mesh=_mesh,
    compiler_params=pltpu.CompilerParams(use_tc_tiling_on_sc=False,
                                          needs_layout_passes=False,
                                          disable_bounds_checks=True),
    scratch_types=[
        pltpu.VMEM((_H, _TAB), jnp.float32),
        pltpu.VMEM((_N + _L,), jnp.float32),
        pltpu.VMEM((_N + _L,), jnp.float32),
        pltpu.VMEM((_N + _L,), jnp.float32),
        pltpu.VMEM((2 * _N_SPATIAL,), jnp.float32),
        pltpu.VMEM((_H, _N), jnp.float32),
        pltpu.VMEM((_H, _N), jnp.float32),
        pltpu.SemaphoreType.DMA,
        pltpu.SemaphoreType.DMA,
    ],
)
def _sc_mask_kernel(coords_t, bias_t, mthr, out, tab, ff, px, py, mt,
                    ob0, ob1, sem0, sem1):
    wid = lax.axis_index("s") * 2 + lax.axis_index("c")
    base = wid * _ROWS_PER_W

    pltpu.sync_copy(bias_t, tab)
    pltpu.sync_copy(coords_t.at[0], ff.at[pl.ds(0, _N)])
    pltpu.sync_copy(coords_t.at[1], px.at[pl.ds(0, _N)])
    pltpu.sync_copy(coords_t.at[2], py.at[pl.ds(0, _N)])
    pltpu.sync_copy(mthr, mt)

    def compute_row(i, ob):
        fi = jnp.full((_L,), ff[pl.ds(i, _L)][0], jnp.float32)
        xi = jnp.full((_L,), px[pl.ds(i, _L)][0], jnp.float32)
        yi = jnp.full((_L,), py[pl.ds(i, _L)][0], jnp.float32)

        @plsc.parallel_loop(0, _VECS, 1, unroll=4)
        def vec_body(v):
            o = v * _L
            xj = px[pl.ds(o, _L)]
            yj = py[pl.ds(o, _L)]
            fj = ff[pl.ds(o, _L)]
            dx = xj - xi
            dy = yj - yi
            d2 = dx * dx + dy * dy
            bits = plsc.bitcast(d2, jnp.int32)
            u = (bits - _FBIAS).astype(jnp.float32) * _INV
            kc = jnp.clip(u.astype(jnp.int32), 0, 29)
            m0 = plsc.load_gather(mt, [kc])
            m1 = plsc.load_gather(mt, [kc + _N_SPATIAL])
            s = kc + ((d2 >= m0).astype(jnp.int32)
                      + (d2 >= m1).astype(jnp.int32))
            t_diff = fj - fi
            yhalf = jnp.clip(t_diff * 0.5, -17.0, 17.0)
            tr = yhalf.astype(jnp.int32)
            ceil = tr + jnp.where(tr.astype(jnp.float32) * 2.0 < t_diff, 1, 0)
            tbin = jnp.clip(ceil + 16, 0, 32)
            idx = s + tbin * _N_SPATIAL
            for h in range(_H):
                ob[h, pl.ds(o, _L)] = plsc.load_gather(tab.at[h], [idx])

    def fire(i, ob, sem):
        for h in range(_H):
            pltpu.async_copy(ob.at[h], out.at[h, i], sem)

    def drain(i, ob, sem):
        for h in range(_H):
            pltpu.make_async_copy(ob.at[h], out.at[h, i], sem).wait()

    compute_row(base, ob0)
    fire(base, ob0, sem0)
    compute_row(base + 1, ob1)
    fire(base + 1, ob1, sem1)

    def pair_body(rr, carry):
        i0 = base + 2 * rr
        drain(i0, ob0, sem0)
        compute_row(i0, ob0)
        fire(i0, ob0, sem0)
        i1 = i0 + 1
        drain(i1, ob1, sem1)
        compute_row(i1, ob1)
        fire(i1, ob1, sem1)
        return carry

    lax.fori_loop(1, _ROWS_PER_W // 2, pair_body, 0)

    drain(base + _ROWS_PER_W - 2, ob0, sem0)
    drain(base + _ROWS_PER_W - 1, ob1, sem1)


_MT_ARR = np.array(_MT0 + _MT1, dtype=np.float32)


def kernel(coords, bias):
    coords_t = coords.T
    bias_t = bias.T
    return _sc_mask_kernel(coords_t, bias_t, jnp.asarray(_MT_ARR))


# speedup vs baseline: 2.5388x; 1.4208x over previous
import functools
import math

import numpy as np
import jax
import jax.numpy as jnp
from jax import lax
from jax.experimental import pallas as pl
from jax.experimental.pallas import tpu as pltpu
from jax.experimental.pallas import tpu_sc as plsc

_N = 2048
_H = 8
_N_SPATIAL = 32
_TAB = (2 * 16 + 1) * _N_SPATIAL
_L = 16
_NW = 32
_ROWS_PER_W = _N // _NW
_RB = 8
_CB = 256
_NBLK = _ROWS_PER_W // _RB
_NCG = _N // _CB
_VPC = _CB // _L


def _spatial_d2_thresholds():
    log_c = jnp.log(jnp.float32(_N_SPATIAL * 8 + 1.0))
    bins = jnp.exp(jnp.linspace(0.0, log_c, _N_SPATIAL))
    b31 = bins[:_N_SPATIAL - 1].astype(jnp.float32)
    bsq = b31 * b31
    xb = jax.lax.bitcast_convert_type(bsq, jnp.int32)
    deltas = jnp.arange(-12, 13, dtype=jnp.int32)
    cand = xb[None, :] + deltas[:, None]
    candf = jax.lax.bitcast_convert_type(cand, jnp.float32)
    ok = jnp.sqrt(candf) > b31[None, :]
    big = jnp.where(ok, cand, jnp.int32(2 ** 31 - 1))
    mbits = jnp.min(big, axis=0)
    return jax.lax.bitcast_convert_type(mbits, jnp.float32)


_PAD = 3.0e38


def _mthresh_table():
    m = _spatial_d2_thresholds()
    pad1 = jnp.full((1,), _PAD, jnp.float32)
    pad66 = jnp.full((66,), _PAD, jnp.float32)
    return jnp.concatenate([m, pad1, m[1:], pad66])
_C = 2.0 * math.log2(257.0) / 31.0
_INV = float(np.float32(1.0 / (_C * 2.0 ** 23)))
_FBIAS = 127 << 23

_mesh = plsc.VectorSubcoreMesh(core_axis_name="c", subcore_axis_name="s")


@functools.partial(
    pl.kernel,
    out_type=jax.ShapeDtypeStruct((_H, _N, _N), jnp.float32),
    mesh=_mesh,
    compiler_params=pltpu.CompilerParams(use_tc_tiling_on_sc=True,
                                          needs_layout_passes=False,
                                          disable_bounds_checks=True),
    scratch_types=[
        pltpu.VMEM((_H * _TAB,), jnp.float32),
        pltpu.VMEM((_N + 2 * _L,), jnp.float32),
        pltpu.VMEM((_N + 2 * _L,), jnp.float32),
        pltpu.VMEM((_N + 2 * _L,), jnp.float32),
        pltpu.VMEM((128,), jnp.float32),
        pltpu.VMEM((2, _H, _RB, _CB), jnp.float32),
        pltpu.SemaphoreType.DMA,
        pltpu.SemaphoreType.DMA,
    ],
)
def _sc_mask_kernel(frames, posx, posy, bias_t, mthr, out,
                    tab, ff, px, py, mt, tb, sem0, sem1):
    wid = lax.axis_index("s") * 2 + lax.axis_index("c")
    base = wid * _ROWS_PER_W
    sems = (sem0, sem1)

    pltpu.sync_copy(bias_t, tab)
    pltpu.sync_copy(frames, ff.at[pl.ds(0, _N)])
    pltpu.sync_copy(posx, px.at[pl.ds(0, _N)])
    pltpu.sync_copy(posy, py.at[pl.ds(0, _N)])
    pltpu.sync_copy(mthr, mt)

    def compute_block(t, par):
        i0 = base + (t >> 3) * _RB
        c0 = (t & 7) * _CB

        def row_body(r, carry):
            i = i0 + r
            fi = jnp.full((_L,), ff[pl.ds(i, _L)][0], jnp.float32)
            xi = jnp.full((_L,), px[pl.ds(i, _L)][0], jnp.float32)
            yi = jnp.full((_L,), py[pl.ds(i, _L)][0], jnp.float32)

            @plsc.parallel_loop(0, _VPC, 1, unroll=4)
            def vec_body(v):
                o = c0 + v * _L
                xj = px[pl.ds(o, _L)]
                yj = py[pl.ds(o, _L)]
                fj = ff[pl.ds(o, _L)]
                dx = xj - xi
                dy = yj - yi
                d2 = dx * dx + dy * dy
                bits = plsc.bitcast(d2, jnp.int32)
                u = (bits - _FBIAS).astype(jnp.float32) * _INV
                kc = jnp.minimum(jnp.maximum(u.astype(jnp.int32), 0), 29)
                m0 = plsc.load_gather(mt, [kc])
                m1 = plsc.load_gather(mt, [kc + _N_SPATIAL])
                s = kc + ((d2 >= m0).astype(jnp.int32)
                          + (d2 >= m1).astype(jnp.int32))
                t_diff = fj - fi
                yhalf = jnp.minimum(jnp.maximum(t_diff * 0.5, -17.0), 17.0)
                tr = yhalf.astype(jnp.int32)
                ceil = tr + jnp.where(
                    tr.astype(jnp.float32) * 2.0 < t_diff, 1, 0)
                tbin = jnp.minimum(jnp.maximum(ceil + 16, 0), 32)
                idx = s + tbin * _N_SPATIAL
                for h in range(_H):
                    tb[par, h, r, pl.ds(v * _L, _L)] = (
                        plsc.load_gather(tab, [idx + h * _TAB]))

            return carry

        lax.fori_loop(0, _RB, row_body, 0)

    def fire(t, par):
        i0 = base + (t >> 3) * _RB
        c0 = (t & 7) * _CB
        for h in range(_H):
            pltpu.async_copy(tb.at[par, h],
                             out.at[h, pl.ds(i0, _RB), pl.ds(c0, _CB)],
                             sems[par])

    def drain(t, par):
        i0 = base + (t >> 3) * _RB
        c0 = (t & 7) * _CB
        for h in range(_H):
            pltpu.make_async_copy(
                tb.at[par, h],
                out.at[h, pl.ds(i0, _RB), pl.ds(c0, _CB)],
                sems[par]).wait()

    n_t = _NBLK * _NCG

    compute_block(0, 0)
    fire(0, 0)
    compute_block(1, 1)
    fire(1, 1)

    def pair_body(tt, carry):
        t0 = 2 * tt
        drain(t0, 0)
        compute_block(t0, 0)
        fire(t0, 0)
        t1 = t0 + 1
        drain(t1, 1)
        compute_block(t1, 1)
        fire(t1, 1)
        return carry

    lax.fori_loop(1, n_t // 2, pair_body, 0)

    drain(n_t - 2, 0)
    drain(n_t - 1, 1)


def kernel(coords, bias):
    frames = coords[:, 0]
    posx = coords[:, 1]
    posy = coords[:, 2]
    bias_t = bias.T.reshape(-1)
    return _sc_mask_kernel(frames, posx, posy, bias_t, _mthresh_table())


# speedup vs baseline: 2.6671x; 1.0505x over previous
import functools
import math

import numpy as np
import jax
import jax.numpy as jnp
from jax import lax
from jax.experimental import pallas as pl
from jax.experimental.pallas import tpu as pltpu
from jax.experimental.pallas import tpu_sc as plsc

_N = 2048
_H = 8
_N_SPATIAL = 32
_TAB = (2 * 16 + 1) * _N_SPATIAL
_L = 16
_NW = 32
_ROWS_PER_W = _N // _NW
_RB = 8
_CB = 512
_NBLK = _ROWS_PER_W // _RB
_NCG = _N // _CB
_VPC = _CB // _L


def _spatial_d2_thresholds():
    log_c = jnp.log(jnp.float32(_N_SPATIAL * 8 + 1.0))
    bins = jnp.exp(jnp.linspace(0.0, log_c, _N_SPATIAL))
    b31 = bins[:_N_SPATIAL - 1].astype(jnp.float32)
    bsq = b31 * b31
    xb = jax.lax.bitcast_convert_type(bsq, jnp.int32)
    deltas = jnp.arange(-12, 13, dtype=jnp.int32)
    cand = xb[None, :] + deltas[:, None]
    candf = jax.lax.bitcast_convert_type(cand, jnp.float32)
    ok = jnp.sqrt(candf) > b31[None, :]
    big = jnp.where(ok, cand, jnp.int32(2 ** 31 - 1))
    mbits = jnp.min(big, axis=0)
    return jax.lax.bitcast_convert_type(mbits, jnp.float32)


_PAD = 3.0e38


def _mthresh_table():
    m = _spatial_d2_thresholds()
    pad1 = jnp.full((1,), _PAD, jnp.float32)
    pad66 = jnp.full((66,), _PAD, jnp.float32)
    return jnp.concatenate([m, pad1, m[1:], pad66])
_C = 2.0 * math.log2(257.0) / 31.0
_INV = float(np.float32(1.0 / (_C * 2.0 ** 23)))
_FBIAS = 127 << 23

_mesh = plsc.VectorSubcoreMesh(core_axis_name="c", subcore_axis_name="s")


@functools.partial(
    pl.kernel,
    out_type=jax.ShapeDtypeStruct((_H, _N, _N), jnp.float32),
    mesh=_mesh,
    compiler_params=pltpu.CompilerParams(use_tc_tiling_on_sc=True,
                                          needs_layout_passes=False,
                                          disable_bounds_checks=True),
    scratch_types=[
        [pltpu.VMEM((_TAB,), jnp.float32) for _ in range(_H)],
        pltpu.VMEM((_N + 2 * _L,), jnp.float32),
        pltpu.VMEM((_N + 2 * _L,), jnp.float32),
        pltpu.VMEM((_N + 2 * _L,), jnp.float32),
        pltpu.VMEM((128,), jnp.float32),
        pltpu.VMEM((2, _H, _RB, _CB), jnp.float32),
        pltpu.SemaphoreType.DMA,
        pltpu.SemaphoreType.DMA,
    ],
)
def _sc_mask_kernel(frames, posx, posy, bias_t, mthr, out,
                    tab, ff, px, py, mt, tb, sem0, sem1):
    wid = lax.axis_index("s") * 2 + lax.axis_index("c")
    base = wid * _ROWS_PER_W
    sems = (sem0, sem1)

    for h in range(_H):
        pltpu.sync_copy(bias_t[h], tab[h])
    pltpu.sync_copy(frames, ff.at[pl.ds(0, _N)])
    pltpu.sync_copy(posx, px.at[pl.ds(0, _N)])
    pltpu.sync_copy(posy, py.at[pl.ds(0, _N)])
    pltpu.sync_copy(mthr, mt)

    def compute_block(t, par):
        i0 = base + (t // _NCG) * _RB
        c0 = (t % _NCG) * _CB

        def row_body(r, carry):
            i = i0 + r
            fi = jnp.full((_L,), ff[pl.ds(i, _L)][0], jnp.float32)
            xi = jnp.full((_L,), px[pl.ds(i, _L)][0], jnp.float32)
            yi = jnp.full((_L,), py[pl.ds(i, _L)][0], jnp.float32)

            @plsc.parallel_loop(0, _VPC, 1, unroll=4)
            def vec_body(v):
                o = c0 + v * _L
                xj = px[pl.ds(o, _L)]
                yj = py[pl.ds(o, _L)]
                fj = ff[pl.ds(o, _L)]
                dx = xj - xi
                dy = yj - yi
                d2 = dx * dx + dy * dy
                bits = plsc.bitcast(d2, jnp.int32)
                u = (bits - _FBIAS).astype(jnp.float32) * _INV
                kc = jnp.minimum(jnp.maximum(u.astype(jnp.int32), 0), 29)
                m0 = plsc.load_gather(mt, [kc])
                m1 = plsc.load_gather(mt, [kc + _N_SPATIAL])
                s = kc + ((d2 >= m0).astype(jnp.int32)
                          + (d2 >= m1).astype(jnp.int32))
                t_diff = fj - fi
                yhalf = jnp.minimum(jnp.maximum(t_diff * 0.5, -17.0), 17.0)
                tr = yhalf.astype(jnp.int32)
                ceil = tr + jnp.where(
                    tr.astype(jnp.float32) * 2.0 < t_diff, 1, 0)
                tbin = jnp.minimum(jnp.maximum(ceil + 16, 0), 32)
                idx = s + tbin * _N_SPATIAL
                for h in range(_H):
                    tb[par, h, r, pl.ds(v * _L, _L)] = (
                        plsc.load_gather(tab[h], [idx]))

            return carry

        lax.fori_loop(0, _RB, row_body, 0)

    def fire(t, par):
        i0 = base + (t // _NCG) * _RB
        c0 = (t % _NCG) * _CB
        for h in range(_H):
            pltpu.async_copy(tb.at[par, h],
                             out.at[h, pl.ds(i0, _RB), pl.ds(c0, _CB)],
                             sems[par])

    def drain(t, par):
        i0 = base + (t // _NCG) * _RB
        c0 = (t % _NCG) * _CB
        for h in range(_H):
            pltpu.make_async_copy(
                tb.at[par, h],
                out.at[h, pl.ds(i0, _RB), pl.ds(c0, _CB)],
                sems[par]).wait()

    n_t = _NBLK * _NCG

    compute_block(0, 0)
    fire(0, 0)
    compute_block(1, 1)
    fire(1, 1)

    def pair_body(tt, carry):
        t0 = 2 * tt
        drain(t0, 0)
        compute_block(t0, 0)
        fire(t0, 0)
        t1 = t0 + 1
        drain(t1, 1)
        compute_block(t1, 1)
        fire(t1, 1)
        return carry

    lax.fori_loop(1, n_t // 2, pair_body, 0)

    drain(n_t - 2, 0)
    drain(n_t - 1, 1)


def kernel(coords, bias):
    frames = coords[:, 0]
    posx = coords[:, 1]
    posy = coords[:, 2]
    bias_t = [bias[:, h] for h in range(_H)]
    return _sc_mask_kernel(frames, posx, posy, bias_t, _mthresh_table())


# speedup vs baseline: 2.6836x; 1.0062x over previous
import functools
import math

import numpy as np
import jax
import jax.numpy as jnp
from jax import lax
from jax.experimental import pallas as pl
from jax.experimental.pallas import tpu as pltpu
from jax.experimental.pallas import tpu_sc as plsc

_N = 2048
_H = 8
_N_SPATIAL = 32
_TAB = (2 * 16 + 1) * _N_SPATIAL
_L = 16
_NW = 32
_ROWS_PER_W = _N // _NW
_RB = 8
_CB = 512
_NBLK = _ROWS_PER_W // _RB
_NCG = _N // _CB
_VPC = _CB // _L


def _spatial_d2_thresholds():
    log_c = jnp.log(jnp.float32(_N_SPATIAL * 8 + 1.0))
    bins = jnp.exp(jnp.linspace(0.0, log_c, _N_SPATIAL))
    b31 = bins[:_N_SPATIAL - 1].astype(jnp.float32)
    bsq = b31 * b31
    xb = jax.lax.bitcast_convert_type(bsq, jnp.int32)
    deltas = jnp.arange(-12, 13, dtype=jnp.int32)
    cand = xb[None, :] + deltas[:, None]
    candf = jax.lax.bitcast_convert_type(cand, jnp.float32)
    ok = jnp.sqrt(candf) > b31[None, :]
    big = jnp.where(ok, cand, jnp.int32(2 ** 31 - 1))
    mbits = jnp.min(big, axis=0)
    return jax.lax.bitcast_convert_type(mbits, jnp.float32)


_PAD = 3.0e38


def _mthresh_table():
    m = _spatial_d2_thresholds()
    pad1 = jnp.full((1,), _PAD, jnp.float32)
    pad66 = jnp.full((66,), _PAD, jnp.float32)
    return jnp.concatenate([m, pad1, m[1:], pad66])
_C = 2.0 * math.log2(257.0) / 31.0
_INV = float(np.float32(1.0 / (_C * 2.0 ** 23)))
_FBIAS = 127 << 23

_mesh = plsc.VectorSubcoreMesh(core_axis_name="c", subcore_axis_name="s")


@functools.partial(
    pl.kernel,
    out_type=jax.ShapeDtypeStruct((_H, _N, _N), jnp.float32),
    mesh=_mesh,
    compiler_params=pltpu.CompilerParams(use_tc_tiling_on_sc=True,
                                          needs_layout_passes=False,
                                          disable_bounds_checks=True),
    scratch_types=[
        [pltpu.VMEM((_TAB,), jnp.float32) for _ in range(_H)],
        pltpu.VMEM((_N + 2 * _L,), jnp.float32),
        pltpu.VMEM((_N + 2 * _L,), jnp.float32),
        pltpu.VMEM((_N + 2 * _L,), jnp.float32),
        pltpu.VMEM((128,), jnp.float32),
        pltpu.VMEM((2, _H, _RB, _CB), jnp.float32),
        pltpu.SemaphoreType.DMA,
        pltpu.SemaphoreType.DMA,
    ],
)
def _sc_mask_kernel(frames, posx, posy, bias_t, mthr, out,
                    tab, ff, px, py, mt, tb, sem0, sem1):
    wid = lax.axis_index("s") * 2 + lax.axis_index("c")
    base = wid * _ROWS_PER_W
    sems = (sem0, sem1)

    for h in range(_H):
        pltpu.sync_copy(bias_t[h], tab[h])
    pltpu.sync_copy(frames, ff.at[pl.ds(0, _N)])
    pltpu.sync_copy(posx, px.at[pl.ds(0, _N)])
    pltpu.sync_copy(posy, py.at[pl.ds(0, _N)])
    pltpu.sync_copy(mthr, mt)

    def compute_block(t, par):
        i0 = base + (t // _NCG) * _RB
        c0 = (t % _NCG) * _CB

        def row_body(r, carry):
            i = i0 + r
            fi = jnp.full((_L,), ff[pl.ds(i, _L)][0], jnp.float32)
            xi = jnp.full((_L,), px[pl.ds(i, _L)][0], jnp.float32)
            yi = jnp.full((_L,), py[pl.ds(i, _L)][0], jnp.float32)

            @plsc.parallel_loop(0, _VPC, 1, unroll=8)
            def vec_body(v):
                o = c0 + v * _L
                xj = px[pl.ds(o, _L)]
                yj = py[pl.ds(o, _L)]
                fj = ff[pl.ds(o, _L)]
                dx = xj - xi
                dy = yj - yi
                d2 = dx * dx + dy * dy
                bits = plsc.bitcast(d2, jnp.int32)
                u = (bits - _FBIAS).astype(jnp.float32) * _INV
                kc = jnp.minimum(jnp.maximum(u.astype(jnp.int32), 0), 29)
                m0 = plsc.load_gather(mt, [kc])
                m1 = plsc.load_gather(mt, [kc + _N_SPATIAL])
                s = kc + ((d2 >= m0).astype(jnp.int32)
                          + (d2 >= m1).astype(jnp.int32))
                t_diff = fj - fi
                yhalf = jnp.minimum(jnp.maximum(t_diff * 0.5, -17.0), 17.0)
                tr = yhalf.astype(jnp.int32)
                ceil = tr + jnp.where(
                    tr.astype(jnp.float32) * 2.0 < t_diff, 1, 0)
                tbin = jnp.minimum(jnp.maximum(ceil + 16, 0), 32)
                idx = s + tbin * _N_SPATIAL
                for h in range(_H):
                    tb[par, h, r, pl.ds(v * _L, _L)] = (
                        plsc.load_gather(tab[h], [idx]))

            return carry

        lax.fori_loop(0, _RB, row_body, 0)

    def fire(t, par):
        i0 = base + (t // _NCG) * _RB
        c0 = (t % _NCG) * _CB
        for h in range(_H):
            pltpu.async_copy(tb.at[par, h],
                             out.at[h, pl.ds(i0, _RB), pl.ds(c0, _CB)],
                             sems[par])

    def drain(t, par):
        i0 = base + (t // _NCG) * _RB
        c0 = (t % _NCG) * _CB
        for h in range(_H):
            pltpu.make_async_copy(
                tb.at[par, h],
                out.at[h, pl.ds(i0, _RB), pl.ds(c0, _CB)],
                sems[par]).wait()

    n_t = _NBLK * _NCG

    compute_block(0, 0)
    fire(0, 0)
    compute_block(1, 1)
    fire(1, 1)

    def pair_body(tt, carry):
        t0 = 2 * tt
        drain(t0, 0)
        compute_block(t0, 0)
        fire(t0, 0)
        t1 = t0 + 1
        drain(t1, 1)
        compute_block(t1, 1)
        fire(t1, 1)
        return carry

    lax.fori_loop(1, n_t // 2, pair_body, 0)

    drain(n_t - 2, 0)
    drain(n_t - 1, 1)


def kernel(coords, bias):
    frames = coords[:, 0]
    posx = coords[:, 1]
    posy = coords[:, 2]
    bias_t = [bias[:, h] for h in range(_H)]
    return _sc_mask_kernel(frames, posx, posy, bias_t, _mthresh_table())
